# baseline (speedup 1.0000x reference)
import functools

import jax
import jax.numpy as jnp
from jax import lax
from jax.experimental import pallas as pl
from jax.experimental.pallas import tpu as pltpu
from jax.experimental.pallas import tpu_sc as plsc

_N = 100000
_T = 95
_L = 16
_NC = 2
_NS = 16
_NW = _NC * _NS

_N_TC = 288 * 128
_N_SC = _N - _N_TC
_CHUNK = 1984
_RT = _N_TC // 128

_mesh = plsc.VectorSubcoreMesh(core_axis_name="c", subcore_axis_name="s")


@functools.partial(
    pl.kernel,
    mesh=_mesh,
    out_type=jax.ShapeDtypeStruct((_N_SC,), jnp.float32),
    compiler_params=pltpu.CompilerParams(needs_layout_passes=False),
    scratch_types=[
        pltpu.VMEM((_T,), jnp.float32),
        pltpu.VMEM((_CHUNK,), jnp.int32),
        pltpu.VMEM((_CHUNK,), jnp.float32),
        pltpu.SemaphoreType.DMA,
        pltpu.SemaphoreType.DMA,
    ],
)
def _atomref_sc(types_hbm, table_hbm, out_hbm, table_v, idx_v, out_v,
                sem_t, sem_i):
    wid = lax.axis_index("s") * _NC + lax.axis_index("c")
    base = lax.min(wid * _CHUNK, _N_SC - _CHUNK)

    ct = pltpu.async_copy(table_hbm, table_v, sem_t)
    ci = pltpu.async_copy(types_hbm.at[pl.ds(base, _CHUNK)], idx_v, sem_i)
    ct.wait()
    ci.wait()

    @plsc.parallel_loop(0, _CHUNK, _L, unroll=4)
    def _body(i):
        idx = idx_v[pl.ds(i, _L)]
        out_v[pl.ds(i, _L)] = plsc.load_gather(table_v, [idx])

    pltpu.sync_copy(out_v, out_hbm.at[pl.ds(base, _CHUNK)])


def _tc_body(tab_ref, idx_ref, out_ref):
    tab = jnp.broadcast_to(tab_ref[...], (_RT, 128))
    out_ref[...] = jnp.take_along_axis(
        tab, idx_ref[...], axis=1, mode="promise_in_bounds")


_atomref_tc = pl.pallas_call(
    _tc_body,
    out_shape=jax.ShapeDtypeStruct((_RT, 128), jnp.float32),
)


def kernel(atom_types, elemental_energies):
    atom_types = atom_types.astype(jnp.int32)
    sc_part = _atomref_sc(atom_types[:_N_SC], elemental_energies)
    table_row = jnp.pad(elemental_energies, (0, 128 - _T)).reshape(1, 128)
    tc_idx = atom_types[_N_SC:].reshape(_RT, 128)
    tc_part = _atomref_tc(table_row, tc_idx).reshape(-1)
    return jnp.concatenate([sc_part, tc_part])


# speedup vs baseline: 1.0919x; 1.0919x over previous
import functools

import jax
import jax.numpy as jnp
from jax import lax
from jax.experimental import pallas as pl
from jax.experimental.pallas import tpu as pltpu
from jax.experimental.pallas import tpu_sc as plsc

_N = 100000
_T = 95
_L = 16
_NC = 2
_NS = 16
_NW = _NC * _NS
_CHUNK = 3136

_mesh = plsc.VectorSubcoreMesh(core_axis_name="c", subcore_axis_name="s")


@functools.partial(
    pl.kernel,
    mesh=_mesh,
    out_type=jax.ShapeDtypeStruct((_N,), jnp.float32),
    compiler_params=pltpu.CompilerParams(needs_layout_passes=False),
    scratch_types=[
        pltpu.VMEM((_T,), jnp.float32),
        pltpu.VMEM((_CHUNK,), jnp.int32),
        pltpu.VMEM((_CHUNK,), jnp.float32),
        pltpu.SemaphoreType.DMA,
        pltpu.SemaphoreType.DMA,
    ],
)
def _atomref_sc(types_hbm, table_hbm, out_hbm, table_v, idx_v, out_v,
                sem_t, sem_i):
    wid = lax.axis_index("s") * _NC + lax.axis_index("c")
    base = lax.min(wid * _CHUNK, _N - _CHUNK)

    ct = pltpu.async_copy(table_hbm, table_v, sem_t)
    ci = pltpu.async_copy(types_hbm.at[pl.ds(base, _CHUNK)], idx_v, sem_i)
    ct.wait()
    ci.wait()

    @plsc.parallel_loop(0, _CHUNK, _L, unroll=8)
    def _body(i):
        idx = idx_v[pl.ds(i, _L)]
        out_v[pl.ds(i, _L)] = plsc.load_gather(table_v, [idx])

    pltpu.sync_copy(out_v, out_hbm.at[pl.ds(base, _CHUNK)])


def kernel(atom_types, elemental_energies):
    return _atomref_sc(atom_types.astype(jnp.int32), elemental_energies)


# speedup vs baseline: 1.1705x; 1.0719x over previous
import functools

import jax
import jax.numpy as jnp
from jax import lax
from jax.experimental import pallas as pl
from jax.experimental.pallas import tpu as pltpu
from jax.experimental.pallas import tpu_sc as plsc

_N = 100000
_T = 95
_L = 16
_NC = 2
_NS = 16
_NW = _NC * _NS
_CHUNK = 6272

_mesh = plsc.VectorSubcoreMesh(core_axis_name="c", subcore_axis_name="s", num_cores=1)


@functools.partial(
    pl.kernel,
    mesh=_mesh,
    out_type=jax.ShapeDtypeStruct((_N,), jnp.float32),
    compiler_params=pltpu.CompilerParams(needs_layout_passes=False),
    scratch_types=[
        pltpu.VMEM((_T,), jnp.float32),
        pltpu.VMEM((_CHUNK,), jnp.int32),
        pltpu.VMEM((_CHUNK,), jnp.float32),
        pltpu.SemaphoreType.DMA,
        pltpu.SemaphoreType.DMA,
    ],
)
def _atomref_sc(types_hbm, table_hbm, out_hbm, table_v, idx_v, out_v,
                sem_t, sem_i):
    wid = lax.axis_index("s")
    base = lax.min(wid * _CHUNK, _N - _CHUNK)

    ct = pltpu.async_copy(table_hbm, table_v, sem_t)
    ci = pltpu.async_copy(types_hbm.at[pl.ds(base, _CHUNK)], idx_v, sem_i)
    ct.wait()
    ci.wait()

    @plsc.parallel_loop(0, _CHUNK, _L, unroll=8)
    def _body(i):
        idx = idx_v[pl.ds(i, _L)]
        out_v[pl.ds(i, _L)] = plsc.load_gather(table_v, [idx])

    pltpu.sync_copy(out_v, out_hbm.at[pl.ds(base, _CHUNK)])


def kernel(atom_types, elemental_energies):
    return _atomref_sc(atom_types.astype(jnp.int32), elemental_energies)
